# baseline (speedup 1.0000x reference)
import functools

import jax
import jax.numpy as jnp
from jax import lax
from jax.experimental import pallas as pl
from jax.experimental.pallas import tpu as pltpu
from jax.experimental.pallas import tpu_sc as plsc

_INFO = plsc.get_sparse_core_info()
_NC, _NS, _LANES = _INFO.num_cores, _INFO.num_subcores, _INFO.num_lanes
_NW = _NC * _NS

_B = 4
_L = 8192
_D = 768
_POSW = _L // _NW
_C = 32
_NCHUNK = _POSW // _C
_VPR = _D // _LANES


@functools.partial(
    pl.kernel,
    out_type=jax.ShapeDtypeStruct((_B * _L, _D), jnp.float32),
    mesh=plsc.VectorSubcoreMesh(core_axis_name="c", subcore_axis_name="s"),
    scratch_types=[
        pltpu.VMEM((_B * _POSW,), jnp.int32),
        pltpu.VMEM((_C, _D), jnp.float32),
        pltpu.VMEM((_C, _D), jnp.float32),
        pltpu.SemaphoreType.DMA,
    ],
)
def _embed(tokens_hbm, table_hbm, pos_hbm, out_hbm, idx_v, pos_v, tok_v, sem):
    cid = lax.axis_index("c")
    sid = lax.axis_index("s")
    wid = sid * _NC + cid

    for b in range(_B):
        pltpu.sync_copy(
            tokens_hbm.at[pl.ds(b * _L + wid * _POSW, _POSW)],
            idx_v.at[pl.ds(b * _POSW, _POSW)],
        )

    def chunk_body(cp, _):
        p0 = wid * _POSW + cp * _C
        pltpu.sync_copy(pos_hbm.at[pl.ds(p0, _C), :], pos_v)

        def batch_body(b, _):
            pltpu.async_copy(
                table_hbm.at[idx_v.at[pl.ds(b * _POSW + cp * _C, _C)]],
                tok_v,
                sem,
            ).wait()

            def add_body(j, _):
                i = j // _VPR
                k = (j % _VPR) * _LANES
                plsc.addupdate(tok_v.at[i, pl.ds(k, _LANES)],
                               pos_v[i, pl.ds(k, _LANES)])
                return 0

            lax.fori_loop(0, _C * _VPR, add_body, 0)

            pltpu.sync_copy(
                tok_v, out_hbm.at[pl.ds(b * _L + p0, _C), :]
            )
            return 0

        lax.fori_loop(0, _B, batch_body, 0)
        return 0

    lax.fori_loop(0, _NCHUNK, chunk_body, 0)


def kernel(tokens, token_table, pos_table):
    B, L = tokens.shape
    flat = tokens.reshape(B * L).astype(jnp.int32)
    out = _embed(flat, token_table, pos_table)
    return out.reshape(B, L, -1)


# speedup vs baseline: 2.9362x; 2.9362x over previous
import functools

import jax
import jax.numpy as jnp
from jax import lax
from jax.experimental import pallas as pl
from jax.experimental.pallas import tpu as pltpu
from jax.experimental.pallas import tpu_sc as plsc

_INFO = plsc.get_sparse_core_info()
_NC, _NS, _LANES = _INFO.num_cores, _INFO.num_subcores, _INFO.num_lanes
_NW = _NC * _NS

_B = 4
_L = 8192
_D = 768
_POSW = _L // _NW
_C = 32
_NPC = _POSW // _C
_NCHUNK = _B * _NPC
_VPR = _D // _LANES


@functools.partial(
    pl.kernel,
    out_type=jax.ShapeDtypeStruct((_B * _L, _D), jnp.float32),
    mesh=plsc.VectorSubcoreMesh(core_axis_name="c", subcore_axis_name="s"),
    scratch_types=[
        pltpu.VMEM((_B * _POSW,), jnp.int32),
        [pltpu.VMEM((_C, _D), jnp.float32) for _ in range(2)],
        [pltpu.VMEM((_C, _D), jnp.float32) for _ in range(2)],
        pltpu.SemaphoreType.DMA((2,)),
        pltpu.SemaphoreType.DMA((2,)),
        pltpu.SemaphoreType.DMA((2,)),
    ],
)
def _embed(tokens_hbm, table_hbm, pos_hbm, out_hbm, idx_v, pos_bufs, tok_bufs,
           psem, gsem, wsem):
    cid = lax.axis_index("c")
    sid = lax.axis_index("s")
    wid = sid * _NC + cid

    for b in range(_B):
        pltpu.sync_copy(
            tokens_hbm.at[pl.ds(b * _L + wid * _POSW, _POSW)],
            idx_v.at[pl.ds(b * _POSW, _POSW)],
        )

    def load_pos(cp, pp):
        p0 = wid * _POSW + cp * _C
        pltpu.async_copy(pos_hbm.at[pl.ds(p0, _C), :], pos_bufs[pp],
                         psem.at[pp])

    def gather(t, jj):
        i0 = (t % 4) * _POSW + (t // 4) * _C
        pltpu.async_copy(table_hbm.at[idx_v.at[pl.ds(i0, _C)]], tok_bufs[jj],
                         gsem.at[jj])

    def out_rows(t):
        return (t % 4) * _L + wid * _POSW + (t // 4) * _C

    def wait_wb(t, jj):
        pltpu.make_async_copy(tok_bufs[jj],
                              out_hbm.at[pl.ds(out_rows(t), _C), :],
                              wsem.at[jj]).wait()

    def wait_pos(pp):
        pltpu.make_async_copy(pos_hbm.at[pl.ds(0, _C), :], pos_bufs[pp],
                              psem.at[pp]).wait()

    def wait_gather(jj):
        pltpu.make_async_copy(table_hbm.at[idx_v.at[pl.ds(0, _C)]],
                              tok_bufs[jj], gsem.at[jj]).wait()

    load_pos(0, 0)
    gather(0, 0)

    def group_body(g, _):
        for j in range(8):
            t = 8 * g + j
            pp = (j // 4) % 2
            jj = j % 2
            jo = (j + 1) % 2

            if j % 4 == 0:
                wait_pos(pp)

                @pl.when(t + 4 < _NCHUNK)
                def _():
                    load_pos(t // 4 + 1, (pp + 1) % 2)

            @pl.when(t + 1 < _NCHUNK)
            def _():
                @pl.when(t >= 1)
                def _():
                    wait_wb(t - 1, jo)
                gather(t + 1, jo)

            wait_gather(jj)

            tok_b = tok_bufs[jj]
            pos_b = pos_bufs[pp]

            def add_row(i, _):
                for k in range(_VPR):
                    plsc.addupdate(tok_b.at[i, pl.ds(k * _LANES, _LANES)],
                                   pos_b[i, pl.ds(k * _LANES, _LANES)])
                return 0

            lax.fori_loop(0, _C, add_row, 0)

            pltpu.async_copy(tok_b, out_hbm.at[pl.ds(out_rows(t), _C), :],
                             wsem.at[jj])
        return 0

    lax.fori_loop(0, _NCHUNK // 8, group_body, 0)

    wait_wb(_NCHUNK - 2, 0)
    wait_wb(_NCHUNK - 1, 1)


def kernel(tokens, token_table, pos_table):
    B, L = tokens.shape
    flat = tokens.reshape(B * L).astype(jnp.int32)
    out = _embed(flat, token_table, pos_table)
    return out.reshape(B, L, -1)


# speedup vs baseline: 3.1979x; 1.0891x over previous
import functools

import jax
import jax.numpy as jnp
from jax import lax
from jax.experimental import pallas as pl
from jax.experimental.pallas import tpu as pltpu
from jax.experimental.pallas import tpu_sc as plsc

_INFO = plsc.get_sparse_core_info()
_NC, _NS, _LANES = _INFO.num_cores, _INFO.num_subcores, _INFO.num_lanes
_NW = _NC * _NS

_B = 4
_L = 8192
_D = 768
_POSW = _L // _NW
_C = 16
_NPC = _POSW // _C
_NCHUNK = _B * _NPC
_VPR = _D // _LANES
_NBUF = 4


@functools.partial(
    pl.kernel,
    out_type=jax.ShapeDtypeStruct((_B * _L, _D), jnp.float32),
    mesh=plsc.VectorSubcoreMesh(core_axis_name="c", subcore_axis_name="s"),
    scratch_types=[
        pltpu.VMEM((_B * _POSW,), jnp.int32),
        [pltpu.VMEM((_C, _D), jnp.float32) for _ in range(2)],
        [pltpu.VMEM((_C, _D), jnp.float32) for _ in range(_NBUF)],
        pltpu.SemaphoreType.DMA((2,)),
        pltpu.SemaphoreType.DMA((_NBUF,)),
        pltpu.SemaphoreType.DMA((_NBUF,)),
    ],
)
def _embed(tokens_hbm, table_hbm, pos_hbm, out_hbm, idx_v, pos_bufs, tok_bufs,
           psem, gsem, wsem):
    cid = lax.axis_index("c")
    sid = lax.axis_index("s")
    wid = sid * _NC + cid

    for b in range(_B):
        pltpu.sync_copy(
            tokens_hbm.at[pl.ds(b * _L + wid * _POSW, _POSW)],
            idx_v.at[pl.ds(b * _POSW, _POSW)],
        )

    def load_pos(cp, pp):
        p0 = wid * _POSW + cp * _C
        pltpu.async_copy(pos_hbm.at[pl.ds(p0, _C), :], pos_bufs[pp],
                         psem.at[pp])

    def gather(t, jj):
        i0 = (t % 4) * _POSW + (t // 4) * _C
        pltpu.async_copy(table_hbm.at[idx_v.at[pl.ds(i0, _C)]], tok_bufs[jj],
                         gsem.at[jj])

    def out_rows(t):
        return (t % 4) * _L + wid * _POSW + (t // 4) * _C

    def wait_wb(t, jj):
        pltpu.make_async_copy(tok_bufs[jj],
                              out_hbm.at[pl.ds(out_rows(t), _C), :],
                              wsem.at[jj]).wait()

    def wait_pos(pp):
        pltpu.make_async_copy(pos_hbm.at[pl.ds(0, _C), :], pos_bufs[pp],
                              psem.at[pp]).wait()

    def wait_gather(jj):
        pltpu.make_async_copy(table_hbm.at[idx_v.at[pl.ds(0, _C)]],
                              tok_bufs[jj], gsem.at[jj]).wait()

    load_pos(0, 0)
    gather(0, 0)
    gather(1, 1)

    def group_body(g, _):
        for j in range(8):
            t = 8 * g + j
            pp = (j // 4) % 2
            jj = j % _NBUF
            jn = (j + 2) % _NBUF

            if j % 4 == 0:
                wait_pos(pp)

                @pl.when(t + 4 < _NCHUNK)
                def _():
                    load_pos(t // 4 + 1, (pp + 1) % 2)

            @pl.when(t + 2 < _NCHUNK)
            def _():
                @pl.when(t >= 2)
                def _():
                    wait_wb(t - 2, jn)
                gather(t + 2, jn)

            wait_gather(jj)

            tok_b = tok_bufs[jj]
            pos_b = pos_bufs[pp]

            def add_row(i, _):
                for k in range(_VPR):
                    plsc.addupdate(tok_b.at[i, pl.ds(k * _LANES, _LANES)],
                                   pos_b[i, pl.ds(k * _LANES, _LANES)])
                return 0

            lax.fori_loop(0, _C, add_row, 0)

            pltpu.async_copy(tok_b, out_hbm.at[pl.ds(out_rows(t), _C), :],
                             wsem.at[jj])
        return 0

    lax.fori_loop(0, _NCHUNK // 8, group_body, 0)

    for t in range(_NCHUNK - 4, _NCHUNK):
        wait_wb(t, t % _NBUF)


def kernel(tokens, token_table, pos_table):
    B, L = tokens.shape
    flat = tokens.reshape(B * L).astype(jnp.int32)
    out = _embed(flat, token_table, pos_table)
    return out.reshape(B, L, -1)


# speedup vs baseline: 3.2718x; 1.0231x over previous
import functools

import jax
import jax.numpy as jnp
from jax import lax
from jax.experimental import pallas as pl
from jax.experimental.pallas import tpu as pltpu
from jax.experimental.pallas import tpu_sc as plsc

_INFO = plsc.get_sparse_core_info()
_NC, _NS, _LANES = _INFO.num_cores, _INFO.num_subcores, _INFO.num_lanes
_NW = _NC * _NS

_B = 4
_L = 8192
_D = 768
_POSW = _L // _NW
_C = 16
_NPC = _POSW // _C
_NCHUNK = _B * _NPC
_VPR = _D // _LANES
_NBUF = 8


@functools.partial(
    pl.kernel,
    out_type=jax.ShapeDtypeStruct((_B * _L, _D), jnp.float32),
    mesh=plsc.VectorSubcoreMesh(core_axis_name="c", subcore_axis_name="s"),
    scratch_types=[
        pltpu.VMEM((_B * _POSW,), jnp.int32),
        [pltpu.VMEM((_C, _D), jnp.float32) for _ in range(2)],
        [pltpu.VMEM((_C, _D), jnp.float32) for _ in range(_NBUF)],
        pltpu.SemaphoreType.DMA((2,)),
        pltpu.SemaphoreType.DMA((_NBUF,)),
        pltpu.SemaphoreType.DMA((_NBUF,)),
    ],
)
def _embed(tokens_hbm, table_hbm, pos_hbm, out_hbm, idx_v, pos_bufs, tok_bufs,
           psem, gsem, wsem):
    cid = lax.axis_index("c")
    sid = lax.axis_index("s")
    wid = sid * _NC + cid

    for b in range(_B):
        pltpu.sync_copy(
            tokens_hbm.at[pl.ds(b * _L + wid * _POSW, _POSW)],
            idx_v.at[pl.ds(b * _POSW, _POSW)],
        )

    def load_pos(cp, pp):
        p0 = wid * _POSW + cp * _C
        pltpu.async_copy(pos_hbm.at[pl.ds(p0, _C), :], pos_bufs[pp],
                         psem.at[pp])

    def gather(t, jj):
        i0 = (t % 4) * _POSW + (t // 4) * _C
        pltpu.async_copy(table_hbm.at[idx_v.at[pl.ds(i0, _C)]], tok_bufs[jj],
                         gsem.at[jj])

    def out_rows(t):
        return (t % 4) * _L + wid * _POSW + (t // 4) * _C

    def wait_wb(t, jj):
        pltpu.make_async_copy(tok_bufs[jj],
                              out_hbm.at[pl.ds(out_rows(t), _C), :],
                              wsem.at[jj]).wait()

    def wait_pos(pp):
        pltpu.make_async_copy(pos_hbm.at[pl.ds(0, _C), :], pos_bufs[pp],
                              psem.at[pp]).wait()

    def wait_gather(jj):
        pltpu.make_async_copy(table_hbm.at[idx_v.at[pl.ds(0, _C)]],
                              tok_bufs[jj], gsem.at[jj]).wait()

    load_pos(0, 0)
    for c0 in range(4):
        gather(c0, c0)

    def group_body(g, _):
        for j in range(8):
            t = 8 * g + j
            pp = (j // 4) % 2
            jj = j % _NBUF
            jn = (j + 4) % _NBUF

            if j % 4 == 0:
                wait_pos(pp)

                @pl.when(t + 4 < _NCHUNK)
                def _():
                    load_pos(t // 4 + 1, (pp + 1) % 2)

            @pl.when(t + 4 < _NCHUNK)
            def _():
                @pl.when(t >= 4)
                def _():
                    wait_wb(t - 4, jn)
                gather(t + 4, jn)

            wait_gather(jj)

            tok_b = tok_bufs[jj]
            pos_b = pos_bufs[pp]

            def add_row(i, _):
                for k in range(_VPR):
                    plsc.addupdate(tok_b.at[i, pl.ds(k * _LANES, _LANES)],
                                   pos_b[i, pl.ds(k * _LANES, _LANES)])
                return 0

            lax.fori_loop(0, _C, add_row, 0)

            pltpu.async_copy(tok_b, out_hbm.at[pl.ds(out_rows(t), _C), :],
                             wsem.at[jj])
        return 0

    lax.fori_loop(0, _NCHUNK // 8, group_body, 0)

    for t in range(_NCHUNK - 8, _NCHUNK):
        wait_wb(t, t % _NBUF)


def kernel(tokens, token_table, pos_table):
    B, L = tokens.shape
    flat = tokens.reshape(B * L).astype(jnp.int32)
    out = _embed(flat, token_table, pos_table)
    return out.reshape(B, L, -1)
